# Initial kernel scaffold; baseline (speedup 1.0000x reference)
import jax
import jax.numpy as jnp
from jax.experimental import pallas as pl


def kernel(X, cluster_assignment):
    raise NotImplementedError("write your pallas kernel here")


# speedup vs baseline: 4.1263x; 4.1263x over previous
import jax
import jax.numpy as jnp
from jax import lax
from jax.experimental import pallas as pl
from jax.experimental.pallas import tpu as pltpu
import jax.experimental.pallas.tpu_sc as plsc
import functools

N_ROWS = 320000
D = 128
N_SEG = 10000
CHUNK = 64
N_CHUNKS = N_ROWS // CHUNK
NC = 2
NS = 16
NW = NC * NS
CHUNKS_PER_W = -(-N_CHUNKS // NW)
SEG_PAD = 10240
SEG_PER_TILE = SEG_PAD // NS
STAGE_STEPS = SEG_PER_TILE // CHUNK
L = 16


def _sc_partial_sums(x, ids, zeros_s, zeros_c1):
    mesh = plsc.VectorSubcoreMesh(core_axis_name="c", subcore_axis_name="s",
                                  num_cores=NC, num_subcores=NS)

    @functools.partial(
        pl.kernel,
        mesh=mesh,
        compiler_params=pltpu.CompilerParams(needs_layout_passes=False),
        out_type=(
            jax.ShapeDtypeStruct((NC * SEG_PAD, D), jnp.float32),
            jax.ShapeDtypeStruct((NW * SEG_PAD,), jnp.float32),
        ),
        scratch_types=[
            pltpu.VMEM_SHARED((SEG_PAD, D), jnp.float32),
            pltpu.VMEM((SEG_PAD,), jnp.float32),
            pltpu.VMEM((CHUNK, D), jnp.float32),
            pltpu.VMEM((CHUNK,), jnp.int32),
        ],
    )
    def k(x_hbm, ids_hbm, zs_hbm, zc1_hbm,
          psum_hbm, pcnt_hbm, acc_sh, cnt_v, rows_v, idx_v):
        cid = lax.axis_index("c")
        sid = lax.axis_index("s")
        w = cid * NS + sid
        tile_base = sid * SEG_PER_TILE

        pltpu.sync_copy(zs_hbm, rows_v)
        for j in range(STAGE_STEPS):
            pltpu.sync_copy(rows_v, acc_sh.at[pl.ds(tile_base + j * CHUNK, CHUNK)])

        pltpu.sync_copy(zc1_hbm, cnt_v)
        plsc.subcore_barrier()

        def body(i, _):
            t = w + NW * i

            @pl.when(t < N_CHUNKS)
            def _():
                base = t * CHUNK
                pltpu.sync_copy(ids_hbm.at[pl.ds(base, CHUNK)], idx_v)
                pltpu.sync_copy(x_hbm.at[pl.ds(base, CHUNK), :], rows_v)
                pltpu.sync_copy(rows_v, acc_sh.at[idx_v], add=True)
                for kq in range(CHUNK // L):
                    v = idx_v[pl.ds(kq * L, L)]
                    plsc.addupdate_scatter(cnt_v, [v], jnp.ones((L,), jnp.float32))
            return 0

        lax.fori_loop(0, CHUNKS_PER_W, body, 0)
        plsc.subcore_barrier()

        off = cid * SEG_PAD + tile_base
        for j in range(STAGE_STEPS):
            pltpu.sync_copy(acc_sh.at[pl.ds(tile_base + j * CHUNK, CHUNK)], rows_v)
            pltpu.sync_copy(rows_v, psum_hbm.at[pl.ds(off + j * CHUNK, CHUNK), :])
        pltpu.sync_copy(cnt_v, pcnt_hbm.at[pl.ds(w * SEG_PAD, SEG_PAD)])

    return k(x, ids, zeros_s, zeros_c1)


def _count_reduce_kernel(pc_ref, out_ref):
    out_ref[...] = jnp.sum(pc_ref[...], axis=0)[:, None]


def _count_reduce(pcnt):
    return pl.pallas_call(
        _count_reduce_kernel,
        out_shape=jax.ShapeDtypeStruct((SEG_PAD, 1), jnp.float32),
    )(pcnt)


def _combine_kernel(ps_ref, c_ref, out_ref):
    s = ps_ref[0] + ps_ref[1]
    out_ref[...] = s / jnp.maximum(c_ref[...], 1.0)


def _combine(psum, cnt):
    B = 1000
    grid = N_SEG // B
    return pl.pallas_call(
        _combine_kernel,
        grid=(grid,),
        in_specs=[
            pl.BlockSpec((NC, B, D), lambda i: (0, i, 0)),
            pl.BlockSpec((B, 1), lambda i: (i, 0)),
        ],
        out_specs=pl.BlockSpec((B, D), lambda i: (i, 0)),
        out_shape=jax.ShapeDtypeStruct((N_SEG, D), jnp.float32),
    )(psum, cnt)


@jax.jit
def kernel(X, cluster_assignment):
    ids = cluster_assignment.astype(jnp.int32)
    zeros_s = jnp.zeros((CHUNK, D), jnp.float32)
    zeros_c1 = jnp.zeros((SEG_PAD,), jnp.float32)
    psum, pcnt = _sc_partial_sums(X, ids, zeros_s, zeros_c1)
    psum = psum.reshape(NC, SEG_PAD, D)
    pcnt = pcnt.reshape(NW, SEG_PAD)
    cnt = _count_reduce(pcnt)
    return _combine(psum, cnt)


# speedup vs baseline: 8.2583x; 2.0014x over previous
import jax
import jax.numpy as jnp
from jax import lax
from jax.experimental import pallas as pl
from jax.experimental.pallas import tpu as pltpu
import jax.experimental.pallas.tpu_sc as plsc
import functools

N_ROWS = 320000
D = 128
N_SEG = 10000
CHUNK = 64
N_CHUNKS = N_ROWS // CHUNK
NC = 2
NS = 16
NW = NC * NS
CHUNKS_PER_W = -(-N_CHUNKS // NW)
SEG_PAD = 10240
SEG_PER_TILE = SEG_PAD // NS
STAGE_STEPS = SEG_PER_TILE // CHUNK
L = 16


def _sc_partial_sums(x, ids, zeros_s, zeros_c1):
    mesh = plsc.VectorSubcoreMesh(core_axis_name="c", subcore_axis_name="s",
                                  num_cores=NC, num_subcores=NS)

    @functools.partial(
        pl.kernel,
        mesh=mesh,
        compiler_params=pltpu.CompilerParams(needs_layout_passes=False),
        out_type=(
            jax.ShapeDtypeStruct((NC * SEG_PAD, D), jnp.float32),
            jax.ShapeDtypeStruct((NW * SEG_PAD,), jnp.float32),
        ),
        scratch_types=[
            pltpu.VMEM_SHARED((SEG_PAD, D), jnp.float32),
            pltpu.VMEM((SEG_PAD,), jnp.float32),
            pltpu.VMEM((2, CHUNK, D), jnp.float32),
            pltpu.VMEM((2, CHUNK), jnp.int32),
            pltpu.SemaphoreType.DMA,
            pltpu.SemaphoreType.DMA,
            pltpu.SemaphoreType.DMA,
            pltpu.SemaphoreType.DMA,
        ],
    )
    def k(x_hbm, ids_hbm, zs_hbm, zc1_hbm,
          psum_hbm, pcnt_hbm, acc_sh, cnt_v, rows2_v, idx2_v,
          sem_i0, sem_i1, sem_r0, sem_r1):
        sem_i = (sem_i0, sem_i1)
        sem_r = (sem_r0, sem_r1)
        cid = lax.axis_index("c")
        sid = lax.axis_index("s")
        w = cid * NS + sid
        tile_base = sid * SEG_PER_TILE

        pltpu.sync_copy(zs_hbm, rows2_v.at[0])
        for j in range(STAGE_STEPS):
            pltpu.sync_copy(rows2_v.at[0], acc_sh.at[pl.ds(tile_base + j * CHUNK, CHUNK)])

        pltpu.sync_copy(zc1_hbm, cnt_v)
        plsc.subcore_barrier()

        def issue(i, b):
            t = w + NW * i

            @pl.when(t < N_CHUNKS)
            def _():
                base = t * CHUNK
                pltpu.async_copy(ids_hbm.at[pl.ds(base, CHUNK)], idx2_v.at[b],
                                 sem_i[b])
                pltpu.async_copy(x_hbm.at[pl.ds(base, CHUNK), :], rows2_v.at[b],
                                 sem_r[b])

        def step(i, b):
            t = w + NW * i

            @pl.when(t < N_CHUNKS)
            def _():
                base = t * CHUNK
                pltpu.make_async_copy(ids_hbm.at[pl.ds(base, CHUNK)],
                                      idx2_v.at[b], sem_i[b]).wait()
                pltpu.make_async_copy(x_hbm.at[pl.ds(base, CHUNK), :],
                                      rows2_v.at[b], sem_r[b]).wait()
                pltpu.sync_copy(rows2_v.at[b], acc_sh.at[idx2_v.at[b]], add=True)
                for kq in range(CHUNK // L):
                    v = idx2_v[b, pl.ds(kq * L, L)]
                    plsc.addupdate_scatter(cnt_v, [v], jnp.ones((L,), jnp.float32))
            issue(i + 2, b)

        issue(0, 0)
        issue(1, 1)

        def body(g, _):
            step(2 * g, 0)
            step(2 * g + 1, 1)
            return 0

        lax.fori_loop(0, (CHUNKS_PER_W + 1) // 2, body, 0)
        plsc.subcore_barrier()

        off = cid * SEG_PAD + tile_base
        for j in range(STAGE_STEPS):
            b = j % 2
            pltpu.sync_copy(acc_sh.at[pl.ds(tile_base + j * CHUNK, CHUNK)],
                            rows2_v.at[b])
            pltpu.sync_copy(rows2_v.at[b], psum_hbm.at[pl.ds(off + j * CHUNK, CHUNK), :])
        pltpu.sync_copy(cnt_v, pcnt_hbm.at[pl.ds(w * SEG_PAD, SEG_PAD)])

    return k(x, ids, zeros_s, zeros_c1)


def _count_reduce_kernel(pc_ref, out_ref):
    out_ref[...] = jnp.sum(pc_ref[...], axis=0)[:, None]


def _count_reduce(pcnt):
    return pl.pallas_call(
        _count_reduce_kernel,
        out_shape=jax.ShapeDtypeStruct((SEG_PAD, 1), jnp.float32),
    )(pcnt)


def _combine_kernel(ps_ref, c_ref, out_ref):
    s = ps_ref[0] + ps_ref[1]
    out_ref[...] = s / jnp.maximum(c_ref[...], 1.0)


def _combine(psum, cnt):
    B = 1000
    grid = N_SEG // B
    return pl.pallas_call(
        _combine_kernel,
        grid=(grid,),
        in_specs=[
            pl.BlockSpec((NC, B, D), lambda i: (0, i, 0)),
            pl.BlockSpec((B, 1), lambda i: (i, 0)),
        ],
        out_specs=pl.BlockSpec((B, D), lambda i: (i, 0)),
        out_shape=jax.ShapeDtypeStruct((N_SEG, D), jnp.float32),
    )(psum, cnt)


@jax.jit
def kernel(X, cluster_assignment):
    ids = cluster_assignment.astype(jnp.int32)
    zeros_s = jnp.zeros((CHUNK, D), jnp.float32)
    zeros_c1 = jnp.zeros((SEG_PAD,), jnp.float32)
    psum, pcnt = _sc_partial_sums(X, ids, zeros_s, zeros_c1)
    psum = psum.reshape(NC, SEG_PAD, D)
    pcnt = pcnt.reshape(NW, SEG_PAD)
    cnt = _count_reduce(pcnt)
    return _combine(psum, cnt)


# speedup vs baseline: 9.5410x; 1.1553x over previous
import jax
import jax.numpy as jnp
from jax import lax
from jax.experimental import pallas as pl
from jax.experimental.pallas import tpu as pltpu
import jax.experimental.pallas.tpu_sc as plsc
import functools

N_ROWS = 320000
D = 128
N_SEG = 10000
CHUNK = 64
N_CHUNKS = N_ROWS // CHUNK
NC = 2
NS = 16
NW = NC * NS
CHUNKS_PER_W = -(-N_CHUNKS // NW)
SEG_PAD = 10240
SEG_PER_TILE = SEG_PAD // NS
STAGE_STEPS = SEG_PER_TILE // CHUNK
L = 16


def _sc_partial_sums(x, ids, zeros_s, zeros_c1):
    mesh = plsc.VectorSubcoreMesh(core_axis_name="c", subcore_axis_name="s",
                                  num_cores=NC, num_subcores=NS)

    @functools.partial(
        pl.kernel,
        mesh=mesh,
        compiler_params=pltpu.CompilerParams(needs_layout_passes=False),
        out_type=(
            jax.ShapeDtypeStruct((NC * SEG_PAD, D), jnp.float32),
            jax.ShapeDtypeStruct((NW * SEG_PAD,), jnp.float32),
        ),
        scratch_types=[
            pltpu.VMEM_SHARED((SEG_PAD, D), jnp.float32),
            pltpu.VMEM((SEG_PAD,), jnp.float32),
            pltpu.VMEM((3, CHUNK, D), jnp.float32),
            pltpu.VMEM((3, CHUNK), jnp.int32),
            pltpu.SemaphoreType.DMA,
            pltpu.SemaphoreType.DMA,
            pltpu.SemaphoreType.DMA,
            pltpu.SemaphoreType.DMA,
            pltpu.SemaphoreType.DMA,
            pltpu.SemaphoreType.DMA,
            pltpu.SemaphoreType.DMA,
            pltpu.SemaphoreType.DMA,
            pltpu.SemaphoreType.DMA,
        ],
    )
    def k(x_hbm, ids_hbm, zs_hbm, zc1_hbm,
          psum_hbm, pcnt_hbm, acc_sh, cnt_v, rows2_v, idx2_v,
          sem_i0, sem_i1, sem_i2, sem_r0, sem_r1, sem_r2,
          sem_s0, sem_s1, sem_s2):
        sem_i = (sem_i0, sem_i1, sem_i2)
        sem_r = (sem_r0, sem_r1, sem_r2)
        sem_s = (sem_s0, sem_s1, sem_s2)
        cid = lax.axis_index("c")
        sid = lax.axis_index("s")
        w = cid * NS + sid
        tile_base = sid * SEG_PER_TILE

        pltpu.sync_copy(zs_hbm, rows2_v.at[0])
        for j in range(STAGE_STEPS):
            pltpu.sync_copy(rows2_v.at[0], acc_sh.at[pl.ds(tile_base + j * CHUNK, CHUNK)])

        pltpu.sync_copy(zc1_hbm, cnt_v)
        plsc.subcore_barrier()

        def issue_gather(i, b):
            t = w + NW * i

            @pl.when(t < N_CHUNKS)
            def _():
                base = t * CHUNK
                pltpu.async_copy(ids_hbm.at[pl.ds(base, CHUNK)], idx2_v.at[b],
                                 sem_i[b])
                pltpu.async_copy(x_hbm.at[pl.ds(base, CHUNK), :], rows2_v.at[b],
                                 sem_r[b])

        def wait_scatter(i, b):
            t = w + NW * i

            @pl.when(jnp.logical_and(i >= 0, t < N_CHUNKS))
            def _():
                pltpu.make_async_copy(rows2_v.at[b], acc_sh.at[idx2_v.at[b]],
                                      sem_s[b]).wait()

        def step(i, b):
            wait_scatter(i - 2, (b + 1) % 3)
            issue_gather(i + 1, (b + 1) % 3)
            t = w + NW * i

            @pl.when(t < N_CHUNKS)
            def _():
                base = t * CHUNK
                pltpu.make_async_copy(ids_hbm.at[pl.ds(base, CHUNK)],
                                      idx2_v.at[b], sem_i[b]).wait()
                pltpu.make_async_copy(x_hbm.at[pl.ds(base, CHUNK), :],
                                      rows2_v.at[b], sem_r[b]).wait()
                pltpu.async_copy(rows2_v.at[b], acc_sh.at[idx2_v.at[b]],
                                 sem_s[b], add=True)
                for kq in range(CHUNK // L):
                    v = idx2_v[b, pl.ds(kq * L, L)]
                    plsc.addupdate_scatter(cnt_v, [v], jnp.ones((L,), jnp.float32))

        issue_gather(0, 0)

        def body(g, _):
            i0 = 3 * g
            step(i0, 0)
            step(i0 + 1, 1)
            step(i0 + 2, 2)
            return 0

        N_G = (CHUNKS_PER_W + 2) // 3 + 1
        lax.fori_loop(0, N_G, body, 0)
        plsc.subcore_barrier()

        off = cid * SEG_PAD + tile_base
        for j in range(STAGE_STEPS):
            b = j % 2
            pltpu.sync_copy(acc_sh.at[pl.ds(tile_base + j * CHUNK, CHUNK)],
                            rows2_v.at[b])
            pltpu.sync_copy(rows2_v.at[b], psum_hbm.at[pl.ds(off + j * CHUNK, CHUNK), :])
        pltpu.sync_copy(cnt_v, pcnt_hbm.at[pl.ds(w * SEG_PAD, SEG_PAD)])

    return k(x, ids, zeros_s, zeros_c1)


def _count_reduce_kernel(pc_ref, out_ref):
    out_ref[...] = jnp.sum(pc_ref[...], axis=0)[:, None]


def _count_reduce(pcnt):
    return pl.pallas_call(
        _count_reduce_kernel,
        out_shape=jax.ShapeDtypeStruct((SEG_PAD, 1), jnp.float32),
    )(pcnt)


def _combine_kernel(ps_ref, c_ref, out_ref):
    s = ps_ref[0] + ps_ref[1]
    out_ref[...] = s / jnp.maximum(c_ref[...], 1.0)


def _combine(psum, cnt):
    B = 1000
    grid = N_SEG // B
    return pl.pallas_call(
        _combine_kernel,
        grid=(grid,),
        in_specs=[
            pl.BlockSpec((NC, B, D), lambda i: (0, i, 0)),
            pl.BlockSpec((B, 1), lambda i: (i, 0)),
        ],
        out_specs=pl.BlockSpec((B, D), lambda i: (i, 0)),
        out_shape=jax.ShapeDtypeStruct((N_SEG, D), jnp.float32),
    )(psum, cnt)


@jax.jit
def kernel(X, cluster_assignment):
    ids = cluster_assignment.astype(jnp.int32)
    zeros_s = jnp.zeros((CHUNK, D), jnp.float32)
    zeros_c1 = jnp.zeros((SEG_PAD,), jnp.float32)
    psum, pcnt = _sc_partial_sums(X, ids, zeros_s, zeros_c1)
    psum = psum.reshape(NC, SEG_PAD, D)
    pcnt = pcnt.reshape(NW, SEG_PAD)
    cnt = _count_reduce(pcnt)
    return _combine(psum, cnt)


# speedup vs baseline: 10.1266x; 1.0614x over previous
import jax
import jax.numpy as jnp
from jax import lax
from jax.experimental import pallas as pl
from jax.experimental.pallas import tpu as pltpu
import jax.experimental.pallas.tpu_sc as plsc
import functools

N_ROWS = 320000
D = 128
N_SEG = 10000
CHUNK = 64
N_CHUNKS = N_ROWS // CHUNK
NC = 2
NS = 16
NW = NC * NS
CHUNKS_PER_W = -(-N_CHUNKS // NW)
SEG_PAD = 10240
SEG_PER_TILE = SEG_PAD // NS
STAGE_STEPS = SEG_PER_TILE // CHUNK
L = 16


def _sc_partial_sums(x, ids, zeros_s, zeros_c1):
    mesh = plsc.VectorSubcoreMesh(core_axis_name="c", subcore_axis_name="s",
                                  num_cores=NC, num_subcores=NS)

    @functools.partial(
        pl.kernel,
        mesh=mesh,
        compiler_params=pltpu.CompilerParams(needs_layout_passes=False),
        out_type=(
            jax.ShapeDtypeStruct((NC * SEG_PAD, D), jnp.float32),
            jax.ShapeDtypeStruct((NW * SEG_PAD,), jnp.float32),
        ),
        scratch_types=[
            pltpu.VMEM_SHARED((SEG_PAD, D), jnp.float32),
            pltpu.VMEM((SEG_PAD,), jnp.float32),
            pltpu.VMEM((3, CHUNK, D), jnp.float32),
            pltpu.VMEM((3, CHUNK), jnp.int32),
            pltpu.SemaphoreType.DMA,
            pltpu.SemaphoreType.DMA,
            pltpu.SemaphoreType.DMA,
            pltpu.SemaphoreType.DMA,
            pltpu.SemaphoreType.DMA,
            pltpu.SemaphoreType.DMA,
            pltpu.SemaphoreType.DMA,
            pltpu.SemaphoreType.DMA,
            pltpu.SemaphoreType.DMA,
        ],
    )
    def k(x_hbm, ids_hbm, zs_hbm, zc1_hbm,
          psum_hbm, pcnt_hbm, acc_sh, cnt_v, rows2_v, idx2_v,
          sem_i0, sem_i1, sem_i2, sem_r0, sem_r1, sem_r2,
          sem_s0, sem_s1, sem_s2):
        sem_i = (sem_i0, sem_i1, sem_i2)
        sem_r = (sem_r0, sem_r1, sem_r2)
        sem_s = (sem_s0, sem_s1, sem_s2)
        cid = lax.axis_index("c")
        sid = lax.axis_index("s")
        w = cid * NS + sid
        tile_base = sid * SEG_PER_TILE

        pltpu.sync_copy(zs_hbm, rows2_v.at[0])
        for j in range(STAGE_STEPS):
            pltpu.sync_copy(rows2_v.at[0], acc_sh.at[pl.ds(tile_base + j * CHUNK, CHUNK)])

        pltpu.sync_copy(zc1_hbm, cnt_v)
        plsc.subcore_barrier()

        def issue_gather(i, b):
            t = w + NW * i

            @pl.when(t < N_CHUNKS)
            def _():
                base = t * CHUNK
                pltpu.async_copy(ids_hbm.at[pl.ds(base, CHUNK)], idx2_v.at[b],
                                 sem_i[b])
                pltpu.async_copy(x_hbm.at[pl.ds(base, CHUNK), :], rows2_v.at[b],
                                 sem_r[b])

        def wait_scatter(i, b):
            t = w + NW * i

            @pl.when(jnp.logical_and(i >= 0, t < N_CHUNKS))
            def _():
                pltpu.make_async_copy(rows2_v.at[b], acc_sh.at[idx2_v.at[b]],
                                      sem_s[b]).wait()

        def step(i, b):
            wait_scatter(i - 2, (b + 1) % 3)
            issue_gather(i + 1, (b + 1) % 3)
            t = w + NW * i

            @pl.when(t < N_CHUNKS)
            def _():
                base = t * CHUNK
                pltpu.make_async_copy(ids_hbm.at[pl.ds(base, CHUNK)],
                                      idx2_v.at[b], sem_i[b]).wait()
                pltpu.make_async_copy(x_hbm.at[pl.ds(base, CHUNK), :],
                                      rows2_v.at[b], sem_r[b]).wait()
                pltpu.async_copy(rows2_v.at[b], acc_sh.at[idx2_v.at[b]],
                                 sem_s[b], add=True)
                for kq in range(CHUNK // L):
                    v = idx2_v[b, pl.ds(kq * L, L)]
                    plsc.addupdate_scatter(cnt_v, [v], jnp.ones((L,), jnp.float32))

        issue_gather(0, 0)

        def body(g, _):
            i0 = 3 * g
            step(i0, 0)
            step(i0 + 1, 1)
            step(i0 + 2, 2)
            return 0

        N_G = (CHUNKS_PER_W + 2) // 3 + 1
        lax.fori_loop(0, N_G, body, 0)
        plsc.subcore_barrier()

        off = cid * SEG_PAD + tile_base
        for j in range(STAGE_STEPS):
            b = j % 2
            pltpu.sync_copy(acc_sh.at[pl.ds(tile_base + j * CHUNK, CHUNK)],
                            rows2_v.at[b])
            pltpu.sync_copy(rows2_v.at[b], psum_hbm.at[pl.ds(off + j * CHUNK, CHUNK), :])
        pltpu.sync_copy(cnt_v, pcnt_hbm.at[pl.ds(w * SEG_PAD, SEG_PAD)])

    return k(x, ids, zeros_s, zeros_c1)


def _combine_kernel(ps_ref, pc_ref, out_ref):
    s = ps_ref[0, :N_SEG] + ps_ref[1, :N_SEG]
    c = jnp.sum(pc_ref[...], axis=0)[:N_SEG, None]
    out_ref[...] = s / jnp.maximum(c, 1.0)


def _combine(psum, pcnt):
    return pl.pallas_call(
        _combine_kernel,
        out_shape=jax.ShapeDtypeStruct((N_SEG, D), jnp.float32),
    )(psum, pcnt)


@jax.jit
def kernel(X, cluster_assignment):
    ids = cluster_assignment.astype(jnp.int32)
    zeros_s = jnp.zeros((CHUNK, D), jnp.float32)
    zeros_c1 = jnp.zeros((SEG_PAD,), jnp.float32)
    psum, pcnt = _sc_partial_sums(X, ids, zeros_s, zeros_c1)
    psum = psum.reshape(NC, SEG_PAD, D)
    pcnt = pcnt.reshape(NW, SEG_PAD)
    return _combine(psum, pcnt)


# speedup vs baseline: 10.3760x; 1.0246x over previous
import jax
import jax.numpy as jnp
from jax import lax
from jax.experimental import pallas as pl
from jax.experimental.pallas import tpu as pltpu
import jax.experimental.pallas.tpu_sc as plsc
import functools

N_ROWS = 320000
D = 128
N_SEG = 10000
CHUNK = 80
N_CHUNKS = N_ROWS // CHUNK
NC = 2
NS = 16
NW = NC * NS
CHUNKS_PER_W = -(-N_CHUNKS // NW)
SEG_PAD = 10240
SEG_PER_TILE = SEG_PAD // NS
STAGE_STEPS = SEG_PER_TILE // CHUNK
L = 16


def _sc_partial_sums(x, ids, zeros_s, zeros_c1):
    mesh = plsc.VectorSubcoreMesh(core_axis_name="c", subcore_axis_name="s",
                                  num_cores=NC, num_subcores=NS)

    @functools.partial(
        pl.kernel,
        mesh=mesh,
        compiler_params=pltpu.CompilerParams(needs_layout_passes=False),
        out_type=(
            jax.ShapeDtypeStruct((NC * SEG_PAD, D), jnp.float32),
            jax.ShapeDtypeStruct((NW * SEG_PAD,), jnp.float32),
        ),
        scratch_types=[
            pltpu.VMEM_SHARED((SEG_PAD, D), jnp.float32),
            pltpu.VMEM((SEG_PAD,), jnp.float32),
            pltpu.VMEM((3, CHUNK, D), jnp.float32),
            pltpu.VMEM((3, CHUNK), jnp.int32),
            pltpu.SemaphoreType.DMA,
            pltpu.SemaphoreType.DMA,
            pltpu.SemaphoreType.DMA,
            pltpu.SemaphoreType.DMA,
            pltpu.SemaphoreType.DMA,
            pltpu.SemaphoreType.DMA,
            pltpu.SemaphoreType.DMA,
            pltpu.SemaphoreType.DMA,
            pltpu.SemaphoreType.DMA,
        ],
    )
    def k(x_hbm, ids_hbm, zs_hbm, zc1_hbm,
          psum_hbm, pcnt_hbm, acc_sh, cnt_v, rows2_v, idx2_v,
          sem_i0, sem_i1, sem_i2, sem_r0, sem_r1, sem_r2,
          sem_s0, sem_s1, sem_s2):
        sem_i = (sem_i0, sem_i1, sem_i2)
        sem_r = (sem_r0, sem_r1, sem_r2)
        sem_s = (sem_s0, sem_s1, sem_s2)
        cid = lax.axis_index("c")
        sid = lax.axis_index("s")
        w = cid * NS + sid
        tile_base = sid * SEG_PER_TILE

        pltpu.sync_copy(zs_hbm, rows2_v.at[0])
        for j in range(STAGE_STEPS):
            pltpu.sync_copy(rows2_v.at[0], acc_sh.at[pl.ds(tile_base + j * CHUNK, CHUNK)])

        pltpu.sync_copy(zc1_hbm, cnt_v)
        plsc.subcore_barrier()

        def issue_gather(i, b):
            t = w + NW * i

            @pl.when(t < N_CHUNKS)
            def _():
                base = t * CHUNK
                pltpu.async_copy(ids_hbm.at[pl.ds(base, CHUNK)], idx2_v.at[b],
                                 sem_i[b])
                pltpu.async_copy(x_hbm.at[pl.ds(base, CHUNK), :], rows2_v.at[b],
                                 sem_r[b])

        def wait_scatter(i, b):
            t = w + NW * i

            @pl.when(jnp.logical_and(i >= 0, t < N_CHUNKS))
            def _():
                pltpu.make_async_copy(rows2_v.at[b], acc_sh.at[idx2_v.at[b]],
                                      sem_s[b]).wait()

        def step(i, b):
            wait_scatter(i - 2, (b + 1) % 3)
            issue_gather(i + 1, (b + 1) % 3)
            t = w + NW * i

            @pl.when(t < N_CHUNKS)
            def _():
                base = t * CHUNK
                pltpu.make_async_copy(ids_hbm.at[pl.ds(base, CHUNK)],
                                      idx2_v.at[b], sem_i[b]).wait()
                pltpu.make_async_copy(x_hbm.at[pl.ds(base, CHUNK), :],
                                      rows2_v.at[b], sem_r[b]).wait()
                pltpu.async_copy(rows2_v.at[b], acc_sh.at[idx2_v.at[b]],
                                 sem_s[b], add=True)
                for kq in range(CHUNK // L):
                    v = idx2_v[b, pl.ds(kq * L, L)]
                    plsc.addupdate_scatter(cnt_v, [v], jnp.ones((L,), jnp.float32))

        issue_gather(0, 0)

        def body(g, _):
            i0 = 3 * g
            step(i0, 0)
            step(i0 + 1, 1)
            step(i0 + 2, 2)
            return 0

        N_G = (CHUNKS_PER_W + 2) // 3 + 1
        lax.fori_loop(0, N_G, body, 0)
        plsc.subcore_barrier()

        off = cid * SEG_PAD + tile_base
        for j in range(STAGE_STEPS):
            b = j % 2
            pltpu.sync_copy(acc_sh.at[pl.ds(tile_base + j * CHUNK, CHUNK)],
                            rows2_v.at[b])
            pltpu.sync_copy(rows2_v.at[b], psum_hbm.at[pl.ds(off + j * CHUNK, CHUNK), :])
        pltpu.sync_copy(cnt_v, pcnt_hbm.at[pl.ds(w * SEG_PAD, SEG_PAD)])

    return k(x, ids, zeros_s, zeros_c1)


def _combine_kernel(ps_ref, pc_ref, out_ref):
    s = ps_ref[0, :N_SEG] + ps_ref[1, :N_SEG]
    c = jnp.sum(pc_ref[...], axis=0)[:N_SEG, None]
    out_ref[...] = s / jnp.maximum(c, 1.0)


def _combine(psum, pcnt):
    return pl.pallas_call(
        _combine_kernel,
        out_shape=jax.ShapeDtypeStruct((N_SEG, D), jnp.float32),
    )(psum, pcnt)


@jax.jit
def kernel(X, cluster_assignment):
    ids = cluster_assignment.astype(jnp.int32)
    zeros_s = jnp.zeros((CHUNK, D), jnp.float32)
    zeros_c1 = jnp.zeros((SEG_PAD,), jnp.float32)
    psum, pcnt = _sc_partial_sums(X, ids, zeros_s, zeros_c1)
    psum = psum.reshape(NC, SEG_PAD, D)
    pcnt = pcnt.reshape(NW, SEG_PAD)
    return _combine(psum, pcnt)


# speedup vs baseline: 10.5791x; 1.0196x over previous
import jax
import jax.numpy as jnp
from jax import lax
from jax.experimental import pallas as pl
from jax.experimental.pallas import tpu as pltpu
import jax.experimental.pallas.tpu_sc as plsc
import functools

N_ROWS = 320000
D = 128
N_SEG = 10000
CHUNK = 80
N_CHUNKS = N_ROWS // CHUNK
NC = 2
NS = 16
NW = NC * NS
CHUNKS_PER_W = -(-N_CHUNKS // NW)
SEG_PAD = 10240
SEG_PER_TILE = SEG_PAD // NS
STAGE_STEPS = SEG_PER_TILE // CHUNK
L = 16


def _sc_partial_sums(x, ids, zeros_s, zeros_c1):
    mesh = plsc.VectorSubcoreMesh(core_axis_name="c", subcore_axis_name="s",
                                  num_cores=NC, num_subcores=NS)

    @functools.partial(
        pl.kernel,
        mesh=mesh,
        compiler_params=pltpu.CompilerParams(needs_layout_passes=False),
        out_type=(
            jax.ShapeDtypeStruct((NC * SEG_PAD, D), jnp.float32),
            jax.ShapeDtypeStruct((NW * SEG_PAD,), jnp.float32),
        ),
        scratch_types=[
            pltpu.VMEM_SHARED((SEG_PAD, D), jnp.float32),
            pltpu.VMEM((SEG_PAD,), jnp.float32),
            pltpu.VMEM((3, CHUNK, D), jnp.float32),
            pltpu.VMEM((3, CHUNK), jnp.int32),
            pltpu.SemaphoreType.DMA,
            pltpu.SemaphoreType.DMA,
            pltpu.SemaphoreType.DMA,
            pltpu.SemaphoreType.DMA,
            pltpu.SemaphoreType.DMA,
            pltpu.SemaphoreType.DMA,
            pltpu.SemaphoreType.DMA,
            pltpu.SemaphoreType.DMA,
            pltpu.SemaphoreType.DMA,
        ],
    )
    def k(x_hbm, ids_hbm, zs_hbm, zc1_hbm,
          psum_hbm, pcnt_hbm, acc_sh, cnt_v, rows2_v, idx2_v,
          sem_i0, sem_i1, sem_i2, sem_r0, sem_r1, sem_r2,
          sem_s0, sem_s1, sem_s2):
        sem_i = (sem_i0, sem_i1, sem_i2)
        sem_r = (sem_r0, sem_r1, sem_r2)
        sem_s = (sem_s0, sem_s1, sem_s2)
        cid = lax.axis_index("c")
        sid = lax.axis_index("s")
        w = cid * NS + sid
        tile_base = sid * SEG_PER_TILE

        pltpu.async_copy(zc1_hbm, cnt_v, sem_s0)
        pltpu.sync_copy(zs_hbm, rows2_v.at[2])

        def issue_gather(i, b):
            t = w + NW * i

            @pl.when(t < N_CHUNKS)
            def _():
                base = t * CHUNK
                pltpu.async_copy(ids_hbm.at[pl.ds(base, CHUNK)], idx2_v.at[b],
                                 sem_i[b])
                pltpu.async_copy(x_hbm.at[pl.ds(base, CHUNK), :], rows2_v.at[b],
                                 sem_r[b])

        def wait_scatter(i, b):
            t = w + NW * i

            @pl.when(jnp.logical_and(i >= 0, t < N_CHUNKS))
            def _():
                pltpu.make_async_copy(rows2_v.at[b], acc_sh.at[idx2_v.at[b]],
                                      sem_s[b]).wait()

        def step(i, b):
            wait_scatter(i - 2, (b + 1) % 3)
            issue_gather(i + 1, (b + 1) % 3)
            t = w + NW * i

            @pl.when(t < N_CHUNKS)
            def _():
                base = t * CHUNK
                pltpu.make_async_copy(ids_hbm.at[pl.ds(base, CHUNK)],
                                      idx2_v.at[b], sem_i[b]).wait()
                pltpu.make_async_copy(x_hbm.at[pl.ds(base, CHUNK), :],
                                      rows2_v.at[b], sem_r[b]).wait()
                pltpu.async_copy(rows2_v.at[b], acc_sh.at[idx2_v.at[b]],
                                 sem_s[b], add=True)
                for kq in range(CHUNK // L):
                    v = idx2_v[b, pl.ds(kq * L, L)]
                    plsc.addupdate_scatter(cnt_v, [v], jnp.ones((L,), jnp.float32))

        issue_gather(0, 0)
        for j in range(STAGE_STEPS):
            pltpu.async_copy(rows2_v.at[2],
                             acc_sh.at[pl.ds(tile_base + j * CHUNK, CHUNK)],
                             sem_s1)
        pltpu.make_async_copy(zc1_hbm, cnt_v, sem_s0).wait()
        for j in range(STAGE_STEPS):
            pltpu.make_async_copy(rows2_v.at[2],
                                  acc_sh.at[pl.ds(tile_base + j * CHUNK, CHUNK)],
                                  sem_s1).wait()
        plsc.subcore_barrier()

        def body(g, _):
            i0 = 3 * g
            step(i0, 0)
            step(i0 + 1, 1)
            step(i0 + 2, 2)
            return 0

        N_G = (CHUNKS_PER_W + 2) // 3 + 1
        lax.fori_loop(0, N_G, body, 0)
        plsc.subcore_barrier()

        off = cid * SEG_PAD + tile_base
        pltpu.async_copy(cnt_v, pcnt_hbm.at[pl.ds(w * SEG_PAD, SEG_PAD)], sem_s2)

        def s2v(j):
            return pltpu.make_async_copy(
                acc_sh.at[pl.ds(tile_base + j * CHUNK, CHUNK)],
                rows2_v.at[j % 2], sem_i[j % 2])

        def v2h(j):
            return pltpu.make_async_copy(
                rows2_v.at[j % 2],
                psum_hbm.at[pl.ds(off + j * CHUNK, CHUNK), :], sem_r[j % 2])

        pltpu.async_copy(acc_sh.at[pl.ds(tile_base, CHUNK)], rows2_v.at[0],
                         sem_i[0])
        for j in range(STAGE_STEPS):
            s2v(j).wait()
            pltpu.async_copy(rows2_v.at[j % 2],
                             psum_hbm.at[pl.ds(off + j * CHUNK, CHUNK), :],
                             sem_r[j % 2])
            if j + 1 < STAGE_STEPS:
                if j - 1 >= 0:
                    v2h(j - 1).wait()
                pltpu.async_copy(
                    acc_sh.at[pl.ds(tile_base + (j + 1) * CHUNK, CHUNK)],
                    rows2_v.at[(j + 1) % 2], sem_i[(j + 1) % 2])
        v2h(STAGE_STEPS - 2).wait()
        v2h(STAGE_STEPS - 1).wait()
        pltpu.make_async_copy(cnt_v, pcnt_hbm.at[pl.ds(w * SEG_PAD, SEG_PAD)],
                              sem_s2).wait()

    return k(x, ids, zeros_s, zeros_c1)


def _combine_kernel(ps_ref, pc_ref, out_ref):
    s = ps_ref[0, :N_SEG] + ps_ref[1, :N_SEG]
    c = jnp.sum(pc_ref[...], axis=0)[:N_SEG, None]
    out_ref[...] = s / jnp.maximum(c, 1.0)


def _combine(psum, pcnt):
    return pl.pallas_call(
        _combine_kernel,
        out_shape=jax.ShapeDtypeStruct((N_SEG, D), jnp.float32),
    )(psum, pcnt)


@jax.jit
def kernel(X, cluster_assignment):
    ids = cluster_assignment.astype(jnp.int32)
    zeros_s = jnp.zeros((CHUNK, D), jnp.float32)
    zeros_c1 = jnp.zeros((SEG_PAD,), jnp.float32)
    psum, pcnt = _sc_partial_sums(X, ids, zeros_s, zeros_c1)
    psum = psum.reshape(NC, SEG_PAD, D)
    pcnt = pcnt.reshape(NW, SEG_PAD)
    return _combine(psum, pcnt)
